# baseline (speedup 1.0000x reference)
import functools

import jax
import jax.numpy as jnp
from jax import lax
from jax.experimental import pallas as pl
from jax.experimental.pallas import tpu as pltpu
from jax.experimental.pallas import tpu_sc as plsc

N = 10000
E = 320000
H = 128
EIN = 16
OUT = 128
NSIG = 15
SIGMAS = [1.5 ** i for i in range(NSIG)]
TW = 144
NC, NS = 2, 16
NW = NC * NS
S = 2
ES = E // S
EW = ES // NW
CH = 40
NCHUNK = EW // CH
NSLOT = 5
NPAD = 10240
RPT = NPAD // NS
TE = 3200
TN = 2000


def _ln(v, g, b, eps=1e-5):
    mu = jnp.mean(v, axis=-1, keepdims=True)
    var = jnp.mean((v - mu) ** 2, axis=-1, keepdims=True)
    return (v - mu) * jax.lax.rsqrt(var + eps) * g + b


def _lrelu(v, s=0.01):
    return jnp.maximum(v, s * v)


def _prep_body(h_ref, xp_ref, wa_ref, wb_ref, p_ref, q_ref):
    hb = h_ref[...]
    xp = xp_ref[...]
    p_ref[...] = jnp.concatenate([jnp.dot(hb, wa_ref[...],
                                          preferred_element_type=jnp.float32), xp], axis=1)
    q_ref[...] = jnp.concatenate([jnp.dot(hb, wb_ref[...],
                                          preferred_element_type=jnp.float32), xp], axis=1)


def _prep(h, xp16, wa, wb):
    grid = (N // TN,)
    return pl.pallas_call(
        _prep_body,
        grid=grid,
        in_specs=[
            pl.BlockSpec((TN, H), lambda i: (i, 0)),
            pl.BlockSpec((TN, 16), lambda i: (i, 0)),
            pl.BlockSpec((H, OUT), lambda i: (0, 0)),
            pl.BlockSpec((H, OUT), lambda i: (0, 0)),
        ],
        out_specs=[
            pl.BlockSpec((TN, TW), lambda i: (i, 0)),
            pl.BlockSpec((TN, TW), lambda i: (i, 0)),
        ],
        out_shape=[
            jax.ShapeDtypeStruct((N, TW), jnp.float32),
            jax.ShapeDtypeStruct((N, TW), jnp.float32),
        ],
    )(h, xp16, wa, wb)


def _gather_body(ptab, qtab, src3, dst3, g1, g2, gx, *rest):
    idx_s, idx_d = rest[0], rest[1]
    pbufs = rest[2:2 + NSLOT]
    qbufs = rest[2 + NSLOT:2 + 2 * NSLOT]
    sems = rest[2 + 2 * NSLOT:2 + 3 * NSLOT]
    cid = lax.axis_index("c")
    sid = lax.axis_index("s")
    wid = sid * NC + cid
    pltpu.sync_copy(src3.at[wid], idx_s)
    pltpu.sync_copy(dst3.at[wid], idx_d)

    def issue(c, j):
        pltpu.async_copy(ptab.at[idx_s.at[c]], pbufs[j], sems[j])
        pltpu.async_copy(qtab.at[idx_d.at[c]], qbufs[j], sems[j])

    for j in range(NSLOT):
        issue(j, j)

    def body(o, _):
        for j in range(NSLOT):
            c = o * NSLOT + j
            base = pl.multiple_of(wid * EW + c * CH, CH)
            pltpu.make_async_copy(ptab.at[idx_s.at[c]], pbufs[j], sems[j]).wait()
            pltpu.make_async_copy(qtab.at[idx_d.at[c]], qbufs[j], sems[j]).wait()
            pltpu.sync_copy(pbufs[j].at[:, pl.ds(0, OUT)], g1.at[pl.ds(base, CH)])
            pltpu.sync_copy(qbufs[j].at[:, pl.ds(0, OUT)], g2.at[pl.ds(base, CH)])
            pltpu.sync_copy(pbufs[j].at[:, pl.ds(OUT, 16)],
                            gx.at[pl.ds(base, CH), pl.ds(0, 16)])
            pltpu.sync_copy(qbufs[j].at[:, pl.ds(OUT, 16)],
                            gx.at[pl.ds(base, CH), pl.ds(16, 16)])

            @pl.when(c + NSLOT < NCHUNK)
            def _():
                issue(c + NSLOT, j)
        return _

    lax.fori_loop(0, NCHUNK // NSLOT, body, None)


def _gather(ptab, qtab, src3, dst3):
    mesh = plsc.VectorSubcoreMesh(core_axis_name="c", subcore_axis_name="s",
                                  num_cores=NC, num_subcores=NS)
    f = pl.kernel(
        _gather_body,
        out_type=[
            jax.ShapeDtypeStruct((ES, OUT), jnp.float32),
            jax.ShapeDtypeStruct((ES, OUT), jnp.float32),
            jax.ShapeDtypeStruct((ES, OUT), jnp.float32),
        ],
        mesh=mesh,
        scratch_types=(
            [pltpu.VMEM((NCHUNK, CH), jnp.int32)] * 2
            + [pltpu.VMEM((CH, TW), jnp.float32)] * (2 * NSLOT)
            + [pltpu.SemaphoreType.DMA] * NSLOT
        ),
        compiler_params=pltpu.CompilerParams(use_tc_tiling_on_sc=False),
    )
    return f(ptab, qtab, src3, dst3)


def _edge_body(invs_ref, g1_ref, g2_ref, gx_ref, eat_ref, wea_ref, wrbf_ref,
               ge1_ref, bte1_ref, we2_ref, be2_ref, wc1_ref, bc1_ref,
               wc2_ref, m_ref, m2t_ref):
    pre = g1_ref[...] + g2_ref[...]
    xr = gx_ref[:, 0:3] - gx_ref[:, 16:19]
    nrm = jnp.sqrt(jnp.sum(xr * xr, axis=1, keepdims=True)) + 1.0
    xr = xr / nrm
    mag = jnp.sum(xr * xr, axis=1, keepdims=True)
    rbf = jnp.exp(-mag * invs_ref[...])
    pre = pre + lax.dot_general(eat_ref[...], wea_ref[...],
                                (((0,), (0,)), ((), ())),
                                preferred_element_type=jnp.float32)
    pre = pre + jnp.dot(rbf, wrbf_ref[...], preferred_element_type=jnp.float32)
    t = _lrelu(pre)
    t = _ln(t, ge1_ref[...], bte1_ref[...])
    m = _lrelu(jnp.dot(t, we2_ref[...], preferred_element_type=jnp.float32) + be2_ref[...])
    c1 = _lrelu(jnp.dot(m, wc1_ref[...], preferred_element_type=jnp.float32) + bc1_ref[...])
    cw = jnp.sum(c1 * wc2_ref[...], axis=1, keepdims=True)
    xupd = xr * cw
    m_ref[...] = m
    xu4 = jnp.concatenate([xupd, jnp.ones((TE, 1), jnp.float32)], axis=1)
    m2t_ref[...] = jnp.concatenate(
        [jnp.transpose(xu4), jnp.zeros((4, TE), jnp.float32)], axis=0)


def _edge_mlp(k, invs, g1, g2, gx, ea_t, wea, wrbf16, ge1, bte1, we2, be2,
              wc1, bc1, wc2r):
    grid = (ES // TE,)
    w0 = lambda i: (0, 0)
    blk = ES // TE
    return pl.pallas_call(
        _edge_body,
        grid=grid,
        in_specs=[
            pl.BlockSpec((1, 16), w0),
            pl.BlockSpec((TE, OUT), lambda i: (i, 0)),
            pl.BlockSpec((TE, OUT), lambda i: (i, 0)),
            pl.BlockSpec((TE, OUT), lambda i: (i, 0)),
            pl.BlockSpec((EIN, TE), lambda i, _k=k, _b=blk: (0, i + _k * _b)),
            pl.BlockSpec((EIN, OUT), w0),
            pl.BlockSpec((16, OUT), w0),
            pl.BlockSpec((1, OUT), w0),
            pl.BlockSpec((1, OUT), w0),
            pl.BlockSpec((OUT, OUT), w0),
            pl.BlockSpec((1, OUT), w0),
            pl.BlockSpec((OUT, OUT), w0),
            pl.BlockSpec((1, OUT), w0),
            pl.BlockSpec((1, OUT), w0),
        ],
        out_specs=[
            pl.BlockSpec((TE, OUT), lambda i: (i, 0)),
            pl.BlockSpec((8, TE), lambda i: (0, i)),
        ],
        out_shape=[
            jax.ShapeDtypeStruct((ES, OUT), jnp.float32),
            jax.ShapeDtypeStruct((8, ES), jnp.float32),
        ],
    )(invs, g1, g2, gx, ea_t, wea, wrbf16, ge1, bte1, we2, be2, wc1, bc1, wc2r)


def _scatter_body(m_hbm, m2t_hbm, dst3, init1, init2, s1, s2, *rest):
    idx = rest[0]
    r128 = rest[1:1 + NSLOT]
    cbufs = rest[1 + NSLOT:1 + 2 * NSLOT]
    acc1, acc2 = rest[1 + 2 * NSLOT], rest[2 + 2 * NSLOT]
    sems = rest[3 + 2 * NSLOT:3 + 3 * NSLOT]
    cid = lax.axis_index("c")
    sid = lax.axis_index("s")
    wid = sid * NC + cid
    row0 = pl.multiple_of(sid * RPT, RPT)
    pltpu.sync_copy(init1.at[cid, pl.ds(row0, RPT)], acc1.at[pl.ds(row0, RPT)])
    pltpu.sync_copy(init2.at[cid, :, pl.ds(row0, RPT)], acc2.at[:, pl.ds(row0, RPT)])
    pltpu.sync_copy(dst3.at[wid], idx)
    plsc.subcore_barrier()

    def issue(c, j):
        base = pl.multiple_of(wid * EW + c * CH, CH)
        pltpu.async_copy(m_hbm.at[pl.ds(base, CH)], r128[j], sems[j])
        pltpu.async_copy(m2t_hbm.at[:, pl.ds(base, CH)], cbufs[j], sems[j])

    for j in range(NSLOT):
        issue(j, j)

    def body(o, _):
        for j in range(NSLOT):
            c = o * NSLOT + j
            base = pl.multiple_of(wid * EW + c * CH, CH)
            pltpu.make_async_copy(m_hbm.at[pl.ds(base, CH)], r128[j], sems[j]).wait()
            pltpu.make_async_copy(m2t_hbm.at[:, pl.ds(base, CH)], cbufs[j],
                                  sems[j]).wait()
            pltpu.sync_copy(r128[j], acc1.at[idx.at[c]], add=True)
            for comp in range(4):
                pltpu.sync_copy(cbufs[j].at[comp], acc2.at[comp].at[idx.at[c]],
                                add=True)

            @pl.when(c + NSLOT < NCHUNK)
            def _():
                issue(c + NSLOT, j)
        return _

    lax.fori_loop(0, NCHUNK // NSLOT, body, None)
    plsc.subcore_barrier()
    pltpu.sync_copy(acc1.at[pl.ds(row0, RPT)], s1.at[cid, pl.ds(row0, RPT)])
    pltpu.sync_copy(acc2.at[:, pl.ds(row0, RPT)], s2.at[cid, :, pl.ds(row0, RPT)])


def _scatter(m, m2t, dst3, init1, init2):
    mesh = plsc.VectorSubcoreMesh(core_axis_name="c", subcore_axis_name="s",
                                  num_cores=NC, num_subcores=NS)
    f = pl.kernel(
        _scatter_body,
        out_type=[
            jax.ShapeDtypeStruct((NC, NPAD, OUT), jnp.float32),
            jax.ShapeDtypeStruct((NC, 4, NPAD), jnp.float32),
        ],
        mesh=mesh,
        scratch_types=(
            [pltpu.VMEM((NCHUNK, CH), jnp.int32)]
            + [pltpu.VMEM((CH, OUT), jnp.float32)] * NSLOT
            + [pltpu.VMEM((8, CH), jnp.float32)] * NSLOT
            + [pltpu.VMEM_SHARED((NPAD, OUT), jnp.float32),
               pltpu.VMEM_SHARED((4, NPAD), jnp.float32)]
            + [pltpu.SemaphoreType.DMA] * NSLOT
        ),
        compiler_params=pltpu.CompilerParams(use_tc_tiling_on_sc=False),
    )
    return f(m, m2t, dst3, init1, init2)


def _node_body(h_ref, orig_ref, xp_ref, s1_ref, s2_ref, wna_ref, wnb_ref,
               wnd_ref, bn1_ref, gn1_ref, btn1_ref, wn2_ref, bn2_ref,
               gnn_ref, bnn_ref, hnew_ref, xnew_ref):
    s = s1_ref[0] + s1_ref[1]
    s2t = s2_ref[0] + s2_ref[1]
    hb = h_ref[...]
    xsum = s2t[:, 0:3]
    deg = jnp.maximum(s2t[:, 3:4], 1.0)
    magg = s / deg
    lnh = _ln(hb, gnn_ref[...], bnn_ref[...])
    t = (jnp.dot(lnh, wna_ref[...], preferred_element_type=jnp.float32)
         + jnp.dot(magg, wnb_ref[...], preferred_element_type=jnp.float32)
         + jnp.dot(orig_ref[...], wnd_ref[...], preferred_element_type=jnp.float32)
         + bn1_ref[...])
    t = _ln(_lrelu(t), gn1_ref[...], btn1_ref[...])
    nu = jnp.dot(t, wn2_ref[...], preferred_element_type=jnp.float32) + bn2_ref[...]
    hnew_ref[...] = 0.75 * nu + 0.25 * hb
    xn = xp_ref[...][:, 0:3] + xsum / deg
    xnew_ref[...] = jnp.concatenate([xn, jnp.zeros((TN, 13), jnp.float32)], axis=1)


def _node_mlp(h, orig, xp16, s1, s2, wna, wnb, wnd, bn1, gn1, btn1, wn2, bn2,
              gnn, bnn):
    grid = (N // TN,)
    w0 = lambda i: (0, 0)
    return pl.pallas_call(
        _node_body,
        grid=grid,
        in_specs=[
            pl.BlockSpec((TN, H), lambda i: (i, 0)),
            pl.BlockSpec((TN, H), lambda i: (i, 0)),
            pl.BlockSpec((TN, 16), lambda i: (i, 0)),
            pl.BlockSpec((NC, TN, OUT), lambda i: (0, i, 0)),
            pl.BlockSpec((NC, TN, 4), lambda i: (0, i, 0)),
            pl.BlockSpec((H, H), w0),
            pl.BlockSpec((H, H), w0),
            pl.BlockSpec((H, H), w0),
            pl.BlockSpec((1, H), w0),
            pl.BlockSpec((1, H), w0),
            pl.BlockSpec((1, H), w0),
            pl.BlockSpec((H, OUT), w0),
            pl.BlockSpec((1, OUT), w0),
            pl.BlockSpec((1, H), w0),
            pl.BlockSpec((1, H), w0),
        ],
        out_specs=[
            pl.BlockSpec((TN, H), lambda i: (i, 0)),
            pl.BlockSpec((TN, 16), lambda i: (i, 0)),
        ],
        out_shape=[
            jax.ShapeDtypeStruct((N, H), jnp.float32),
            jax.ShapeDtypeStruct((N, 16), jnp.float32),
        ],
    )(h, orig, xp16, s1, s2, wna, wnb, wnd, bn1, gn1, btn1, wn2, bn2, gnn, bnn)


def kernel(h, x, orig_node_feats, edge_attr, edge_index,
           We1, be1, ge1, bte1, We2, be2,
           Wn1, bn1, gn1, btn1, Wn2, bn2,
           Wc1, bc1, Wc2, g_nn, b_nn):
    src4 = edge_index[0].astype(jnp.int32).reshape(S, NW, NCHUNK, CH)
    dst4 = edge_index[1].astype(jnp.int32).reshape(S, NW, NCHUNK, CH)
    xp16 = jnp.pad(x, ((0, 0), (0, 13)))
    ea_t = edge_attr.T

    We1a = We1[:H]
    We1b = We1[H:2 * H]
    Wea = We1[2 * H:2 * H + EIN]
    Wrbf16 = jnp.concatenate([We1[2 * H + EIN:], be1[None, :]], axis=0)
    invs = jnp.array([1.0 / s for s in SIGMAS] + [0.0], jnp.float32)[None, :]

    ptab, qtab = _prep(h, xp16, We1a, We1b)
    s1 = jnp.zeros((NC, NPAD, OUT), jnp.float32)
    s2 = jnp.zeros((NC, 4, NPAD), jnp.float32)
    for k in range(S):
        g1, g2, gx = _gather(ptab, qtab, src4[k], dst4[k])
        m, m2 = _edge_mlp(k, invs, g1, g2, gx, ea_t,
                          Wea, Wrbf16, ge1[None, :], bte1[None, :],
                          We2, be2[None, :], Wc1, bc1[None, :], Wc2.T)
        s1, s2 = _scatter(m, m2, dst4[k], s1, s2)
    s2r = jnp.transpose(s2, (0, 2, 1))
    h_new, xnew16 = _node_mlp(h, orig_node_feats, xp16, s1, s2r,
                              Wn1[:H], Wn1[H:2 * H], Wn1[3 * H:],
                              bn1[None, :], gn1[None, :], btn1[None, :],
                              Wn2, bn2[None, :], g_nn[None, :], b_nn[None, :])
    return (h_new, xnew16[:, 0:3])


# speedup vs baseline: 1.0232x; 1.0232x over previous
import functools

import jax
import jax.numpy as jnp
from jax import lax
from jax.experimental import pallas as pl
from jax.experimental.pallas import tpu as pltpu
from jax.experimental.pallas import tpu_sc as plsc

N = 10000
E = 320000
H = 128
EIN = 16
OUT = 128
NSIG = 15
SIGMAS = [1.5 ** i for i in range(NSIG)]
TW = 144
NC, NS = 2, 16
NW = NC * NS
S = 5
ES = E // S
EW = ES // NW
CH = 40
NCHUNK = EW // CH
NSLOT = 5
NPAD = 10240
RPT = NPAD // NS
TE = 3200
TN = 2000


def _ln(v, g, b, eps=1e-5):
    mu = jnp.mean(v, axis=-1, keepdims=True)
    var = jnp.mean((v - mu) ** 2, axis=-1, keepdims=True)
    return (v - mu) * jax.lax.rsqrt(var + eps) * g + b


def _lrelu(v, s=0.01):
    return jnp.maximum(v, s * v)


def _prep_body(h_ref, xp_ref, wa_ref, wb_ref, p_ref, q_ref):
    hb = h_ref[...]
    xp = xp_ref[...]
    p_ref[...] = jnp.concatenate([jnp.dot(hb, wa_ref[...],
                                          preferred_element_type=jnp.float32), xp], axis=1)
    q_ref[...] = jnp.concatenate([jnp.dot(hb, wb_ref[...],
                                          preferred_element_type=jnp.float32), xp], axis=1)


def _prep(h, xp16, wa, wb):
    grid = (N // TN,)
    return pl.pallas_call(
        _prep_body,
        grid=grid,
        in_specs=[
            pl.BlockSpec((TN, H), lambda i: (i, 0)),
            pl.BlockSpec((TN, 16), lambda i: (i, 0)),
            pl.BlockSpec((H, OUT), lambda i: (0, 0)),
            pl.BlockSpec((H, OUT), lambda i: (0, 0)),
        ],
        out_specs=[
            pl.BlockSpec((TN, TW), lambda i: (i, 0)),
            pl.BlockSpec((TN, TW), lambda i: (i, 0)),
        ],
        out_shape=[
            jax.ShapeDtypeStruct((N, TW), jnp.float32),
            jax.ShapeDtypeStruct((N, TW), jnp.float32),
        ],
    )(h, xp16, wa, wb)


def _gather_body(ptab, qtab, src3, dst3, g1, g2, gx, *rest):
    idx_s, idx_d = rest[0], rest[1]
    pbufs = rest[2:2 + NSLOT]
    qbufs = rest[2 + NSLOT:2 + 2 * NSLOT]
    sems = rest[2 + 2 * NSLOT:2 + 3 * NSLOT]
    cid = lax.axis_index("c")
    sid = lax.axis_index("s")
    wid = sid * NC + cid
    pltpu.sync_copy(src3.at[wid], idx_s)
    pltpu.sync_copy(dst3.at[wid], idx_d)

    def issue(c, j):
        pltpu.async_copy(ptab.at[idx_s.at[c]], pbufs[j], sems[j])
        pltpu.async_copy(qtab.at[idx_d.at[c]], qbufs[j], sems[j])

    for j in range(NSLOT):
        issue(j, j)

    def body(o, _):
        for j in range(NSLOT):
            c = o * NSLOT + j
            base = pl.multiple_of(wid * EW + c * CH, CH)
            pltpu.make_async_copy(ptab.at[idx_s.at[c]], pbufs[j], sems[j]).wait()
            pltpu.make_async_copy(qtab.at[idx_d.at[c]], qbufs[j], sems[j]).wait()
            pltpu.sync_copy(pbufs[j].at[:, pl.ds(0, OUT)], g1.at[pl.ds(base, CH)])
            pltpu.sync_copy(qbufs[j].at[:, pl.ds(0, OUT)], g2.at[pl.ds(base, CH)])
            pltpu.sync_copy(pbufs[j].at[:, pl.ds(OUT, 16)],
                            gx.at[pl.ds(base, CH), pl.ds(0, 16)])
            pltpu.sync_copy(qbufs[j].at[:, pl.ds(OUT, 16)],
                            gx.at[pl.ds(base, CH), pl.ds(16, 16)])

            @pl.when(c + NSLOT < NCHUNK)
            def _():
                issue(c + NSLOT, j)
        return _

    lax.fori_loop(0, NCHUNK // NSLOT, body, None)


def _gather(ptab, qtab, src3, dst3):
    mesh = plsc.VectorSubcoreMesh(core_axis_name="c", subcore_axis_name="s",
                                  num_cores=NC, num_subcores=NS)
    f = pl.kernel(
        _gather_body,
        out_type=[
            jax.ShapeDtypeStruct((ES, OUT), jnp.float32),
            jax.ShapeDtypeStruct((ES, OUT), jnp.float32),
            jax.ShapeDtypeStruct((ES, OUT), jnp.float32),
        ],
        mesh=mesh,
        scratch_types=(
            [pltpu.VMEM((NCHUNK, CH), jnp.int32)] * 2
            + [pltpu.VMEM((CH, TW), jnp.float32)] * (2 * NSLOT)
            + [pltpu.SemaphoreType.DMA] * NSLOT
        ),
        compiler_params=pltpu.CompilerParams(use_tc_tiling_on_sc=False),
    )
    return f(ptab, qtab, src3, dst3)


def _edge_body(invs_ref, g1_ref, g2_ref, gx_ref, eat_ref, wea_ref, wrbf_ref,
               ge1_ref, bte1_ref, we2_ref, be2_ref, wc1_ref, bc1_ref,
               wc2_ref, m_ref, m2_ref):
    pre = g1_ref[...] + g2_ref[...]
    xr = gx_ref[:, 0:3] - gx_ref[:, 16:19]
    nrm = jnp.sqrt(jnp.sum(xr * xr, axis=1, keepdims=True)) + 1.0
    xr = xr / nrm
    mag = jnp.sum(xr * xr, axis=1, keepdims=True)
    rbf = jnp.exp(-mag * invs_ref[...])
    pre = pre + lax.dot_general(eat_ref[...], wea_ref[...],
                                (((0,), (0,)), ((), ())),
                                preferred_element_type=jnp.float32)
    pre = pre + jnp.dot(rbf, wrbf_ref[...], preferred_element_type=jnp.float32)
    t = _lrelu(pre)
    t = _ln(t, ge1_ref[...], bte1_ref[...])
    m = _lrelu(jnp.dot(t, we2_ref[...], preferred_element_type=jnp.float32) + be2_ref[...])
    c1 = _lrelu(jnp.dot(m, wc1_ref[...], preferred_element_type=jnp.float32) + bc1_ref[...])
    cw = jnp.sum(c1 * wc2_ref[...], axis=1, keepdims=True)
    xupd = xr * cw
    ones = jnp.ones((TE, 1), jnp.float32)
    pad = jnp.zeros((TE, OUT - 4), jnp.float32)
    m_ref[...] = m
    m2_ref[...] = jnp.concatenate([xupd, ones, pad], axis=1)


def _edge_mlp(k, invs, g1, g2, gx, ea_t, wea, wrbf16, ge1, bte1, we2, be2,
              wc1, bc1, wc2r):
    grid = (ES // TE,)
    w0 = lambda i: (0, 0)
    blk = ES // TE
    return pl.pallas_call(
        _edge_body,
        grid=grid,
        in_specs=[
            pl.BlockSpec((1, 16), w0),
            pl.BlockSpec((TE, OUT), lambda i: (i, 0)),
            pl.BlockSpec((TE, OUT), lambda i: (i, 0)),
            pl.BlockSpec((TE, OUT), lambda i: (i, 0)),
            pl.BlockSpec((EIN, TE), lambda i, _k=k, _b=blk: (0, i + _k * _b)),
            pl.BlockSpec((EIN, OUT), w0),
            pl.BlockSpec((16, OUT), w0),
            pl.BlockSpec((1, OUT), w0),
            pl.BlockSpec((1, OUT), w0),
            pl.BlockSpec((OUT, OUT), w0),
            pl.BlockSpec((1, OUT), w0),
            pl.BlockSpec((OUT, OUT), w0),
            pl.BlockSpec((1, OUT), w0),
            pl.BlockSpec((1, OUT), w0),
        ],
        out_specs=[
            pl.BlockSpec((TE, OUT), lambda i: (i, 0)),
            pl.BlockSpec((TE, OUT), lambda i: (i, 0)),
        ],
        out_shape=[
            jax.ShapeDtypeStruct((ES, OUT), jnp.float32),
            jax.ShapeDtypeStruct((ES, OUT), jnp.float32),
        ],
    )(invs, g1, g2, gx, ea_t, wea, wrbf16, ge1, bte1, we2, be2, wc1, bc1, wc2r)


def _scatter_body(m_hbm, m2_hbm, dst3, init1, init2, s1, s2, *rest):
    idx = rest[0]
    r128 = rest[1:1 + NSLOT]
    r16 = rest[1 + NSLOT:1 + 2 * NSLOT]
    acc1, acc2 = rest[1 + 2 * NSLOT], rest[2 + 2 * NSLOT]
    sems = rest[3 + 2 * NSLOT:3 + 3 * NSLOT]
    cid = lax.axis_index("c")
    sid = lax.axis_index("s")
    wid = sid * NC + cid
    row0 = pl.multiple_of(sid * RPT, RPT)
    pltpu.sync_copy(init1.at[cid, pl.ds(row0, RPT)], acc1.at[pl.ds(row0, RPT)])
    pltpu.sync_copy(init2.at[cid, pl.ds(row0, RPT)], acc2.at[pl.ds(row0, RPT)])
    pltpu.sync_copy(dst3.at[wid], idx)
    plsc.subcore_barrier()

    def issue(c, j):
        base = pl.multiple_of(wid * EW + c * CH, CH)
        pltpu.async_copy(m_hbm.at[pl.ds(base, CH)], r128[j], sems[j])
        pltpu.async_copy(m2_hbm.at[pl.ds(base, CH), pl.ds(0, 16)], r16[j], sems[j])

    for j in range(NSLOT):
        issue(j, j)

    def body(o, _):
        for j in range(NSLOT):
            c = o * NSLOT + j
            base = pl.multiple_of(wid * EW + c * CH, CH)
            pltpu.make_async_copy(m_hbm.at[pl.ds(base, CH)], r128[j], sems[j]).wait()
            pltpu.make_async_copy(m2_hbm.at[pl.ds(base, CH), pl.ds(0, 16)],
                                  r16[j], sems[j]).wait()
            pltpu.sync_copy(r128[j], acc1.at[idx.at[c]], add=True)
            pltpu.sync_copy(r16[j], acc2.at[idx.at[c]], add=True)

            @pl.when(c + NSLOT < NCHUNK)
            def _():
                issue(c + NSLOT, j)
        return _

    lax.fori_loop(0, NCHUNK // NSLOT, body, None)
    plsc.subcore_barrier()
    pltpu.sync_copy(acc1.at[pl.ds(row0, RPT)], s1.at[cid, pl.ds(row0, RPT)])
    pltpu.sync_copy(acc2.at[pl.ds(row0, RPT)], s2.at[cid, pl.ds(row0, RPT)])


def _scatter(m, m2, dst3, init1, init2):
    mesh = plsc.VectorSubcoreMesh(core_axis_name="c", subcore_axis_name="s",
                                  num_cores=NC, num_subcores=NS)
    f = pl.kernel(
        _scatter_body,
        out_type=[
            jax.ShapeDtypeStruct((NC, NPAD, OUT), jnp.float32),
            jax.ShapeDtypeStruct((NC, NPAD, 16), jnp.float32),
        ],
        mesh=mesh,
        scratch_types=(
            [pltpu.VMEM((NCHUNK, CH), jnp.int32)]
            + [pltpu.VMEM((CH, OUT), jnp.float32)] * NSLOT
            + [pltpu.VMEM((CH, 16), jnp.float32)] * NSLOT
            + [pltpu.VMEM_SHARED((NPAD, OUT), jnp.float32),
               pltpu.VMEM_SHARED((NPAD, 16), jnp.float32)]
            + [pltpu.SemaphoreType.DMA] * NSLOT
        ),
        compiler_params=pltpu.CompilerParams(use_tc_tiling_on_sc=False),
    )
    return f(m, m2, dst3, init1, init2)


def _node_body(h_ref, orig_ref, xp_ref, s1_ref, s2_ref, wna_ref, wnb_ref,
               wnd_ref, bn1_ref, gn1_ref, btn1_ref, wn2_ref, bn2_ref,
               gnn_ref, bnn_ref, hnew_ref, xnew_ref):
    s = s1_ref[0] + s1_ref[1]
    s2 = s2_ref[0] + s2_ref[1]
    hb = h_ref[...]
    xsum = s2[:, 0:3]
    deg = jnp.maximum(s2[:, 3:4], 1.0)
    magg = s / deg
    lnh = _ln(hb, gnn_ref[...], bnn_ref[...])
    t = (jnp.dot(lnh, wna_ref[...], preferred_element_type=jnp.float32)
         + jnp.dot(magg, wnb_ref[...], preferred_element_type=jnp.float32)
         + jnp.dot(orig_ref[...], wnd_ref[...], preferred_element_type=jnp.float32)
         + bn1_ref[...])
    t = _ln(_lrelu(t), gn1_ref[...], btn1_ref[...])
    nu = jnp.dot(t, wn2_ref[...], preferred_element_type=jnp.float32) + bn2_ref[...]
    hnew_ref[...] = 0.75 * nu + 0.25 * hb
    xn = xp_ref[...][:, 0:3] + xsum / deg
    xnew_ref[...] = jnp.concatenate([xn, jnp.zeros((TN, 13), jnp.float32)], axis=1)


def _node_mlp(h, orig, xp16, s1, s2, wna, wnb, wnd, bn1, gn1, btn1, wn2, bn2,
              gnn, bnn):
    grid = (N // TN,)
    w0 = lambda i: (0, 0)
    return pl.pallas_call(
        _node_body,
        grid=grid,
        in_specs=[
            pl.BlockSpec((TN, H), lambda i: (i, 0)),
            pl.BlockSpec((TN, H), lambda i: (i, 0)),
            pl.BlockSpec((TN, 16), lambda i: (i, 0)),
            pl.BlockSpec((NC, TN, OUT), lambda i: (0, i, 0)),
            pl.BlockSpec((NC, TN, 16), lambda i: (0, i, 0)),
            pl.BlockSpec((H, H), w0),
            pl.BlockSpec((H, H), w0),
            pl.BlockSpec((H, H), w0),
            pl.BlockSpec((1, H), w0),
            pl.BlockSpec((1, H), w0),
            pl.BlockSpec((1, H), w0),
            pl.BlockSpec((H, OUT), w0),
            pl.BlockSpec((1, OUT), w0),
            pl.BlockSpec((1, H), w0),
            pl.BlockSpec((1, H), w0),
        ],
        out_specs=[
            pl.BlockSpec((TN, H), lambda i: (i, 0)),
            pl.BlockSpec((TN, 16), lambda i: (i, 0)),
        ],
        out_shape=[
            jax.ShapeDtypeStruct((N, H), jnp.float32),
            jax.ShapeDtypeStruct((N, 16), jnp.float32),
        ],
    )(h, orig, xp16, s1, s2, wna, wnb, wnd, bn1, gn1, btn1, wn2, bn2, gnn, bnn)


def kernel(h, x, orig_node_feats, edge_attr, edge_index,
           We1, be1, ge1, bte1, We2, be2,
           Wn1, bn1, gn1, btn1, Wn2, bn2,
           Wc1, bc1, Wc2, g_nn, b_nn):
    src4 = edge_index[0].astype(jnp.int32).reshape(S, NW, NCHUNK, CH)
    dst4 = edge_index[1].astype(jnp.int32).reshape(S, NW, NCHUNK, CH)
    xp16 = jnp.pad(x, ((0, 0), (0, 13)))
    ea_t = edge_attr.T

    We1a = We1[:H]
    We1b = We1[H:2 * H]
    Wea = We1[2 * H:2 * H + EIN]
    Wrbf16 = jnp.concatenate([We1[2 * H + EIN:], be1[None, :]], axis=0)
    invs = jnp.array([1.0 / s for s in SIGMAS] + [0.0], jnp.float32)[None, :]

    ptab, qtab = _prep(h, xp16, We1a, We1b)
    s1 = jnp.zeros((NC, NPAD, OUT), jnp.float32)
    s2 = jnp.zeros((NC, NPAD, 16), jnp.float32)
    for k in range(S):
        g1, g2, gx = _gather(ptab, qtab, src4[k], dst4[k])
        m, m2 = _edge_mlp(k, invs, g1, g2, gx, ea_t,
                          Wea, Wrbf16, ge1[None, :], bte1[None, :],
                          We2, be2[None, :], Wc1, bc1[None, :], Wc2.T)
        s1, s2 = _scatter(m, m2, dst4[k], s1, s2)
    h_new, xnew16 = _node_mlp(h, orig_node_feats, xp16, s1, s2,
                              Wn1[:H], Wn1[H:2 * H], Wn1[3 * H:],
                              bn1[None, :], gn1[None, :], btn1[None, :],
                              Wn2, bn2[None, :], g_nn[None, :], b_nn[None, :])
    return (h_new, xnew16[:, 0:3])


# speedup vs baseline: 1.0455x; 1.0218x over previous
import functools

import jax
import jax.numpy as jnp
from jax import lax
from jax.experimental import pallas as pl
from jax.experimental.pallas import tpu as pltpu
from jax.experimental.pallas import tpu_sc as plsc

N = 10000
E = 320000
H = 128
EIN = 16
OUT = 128
NSIG = 15
SIGMAS = [1.5 ** i for i in range(NSIG)]
TW = 144
NC, NS = 2, 16
NW = NC * NS
S = 2
ES = E // S
EW = ES // NW
CH = 40
NCHUNK = EW // CH
NSLOT = 5
NPAD = 10240
RPT = NPAD // NS
TE = 6400
TN = 2000


def _ln(v, g, b, eps=1e-5):
    mu = jnp.mean(v, axis=-1, keepdims=True)
    var = jnp.mean((v - mu) ** 2, axis=-1, keepdims=True)
    return (v - mu) * jax.lax.rsqrt(var + eps) * g + b


def _lrelu(v, s=0.01):
    return jnp.maximum(v, s * v)


def _prep_body(h_ref, xp_ref, wa_ref, wb_ref, p_ref, q_ref):
    hb = h_ref[...]
    xp = xp_ref[...]
    p_ref[...] = jnp.concatenate([jnp.dot(hb, wa_ref[...],
                                          preferred_element_type=jnp.float32), xp], axis=1)
    q_ref[...] = jnp.concatenate([jnp.dot(hb, wb_ref[...],
                                          preferred_element_type=jnp.float32), xp], axis=1)


def _prep(h, xp16, wa, wb):
    grid = (N // TN,)
    return pl.pallas_call(
        _prep_body,
        grid=grid,
        in_specs=[
            pl.BlockSpec((TN, H), lambda i: (i, 0)),
            pl.BlockSpec((TN, 16), lambda i: (i, 0)),
            pl.BlockSpec((H, OUT), lambda i: (0, 0)),
            pl.BlockSpec((H, OUT), lambda i: (0, 0)),
        ],
        out_specs=[
            pl.BlockSpec((TN, TW), lambda i: (i, 0)),
            pl.BlockSpec((TN, TW), lambda i: (i, 0)),
        ],
        out_shape=[
            jax.ShapeDtypeStruct((N, TW), jnp.float32),
            jax.ShapeDtypeStruct((N, TW), jnp.float32),
        ],
    )(h, xp16, wa, wb)


def _gather_body(ptab, qtab, src3, dst3, g1, g2, gx, *rest):
    idx_s, idx_d = rest[0], rest[1]
    pbufs = rest[2:2 + NSLOT]
    qbufs = rest[2 + NSLOT:2 + 2 * NSLOT]
    sems = rest[2 + 2 * NSLOT:2 + 3 * NSLOT]
    cid = lax.axis_index("c")
    sid = lax.axis_index("s")
    wid = sid * NC + cid
    pltpu.sync_copy(src3.at[wid], idx_s)
    pltpu.sync_copy(dst3.at[wid], idx_d)

    def issue(c, j):
        pltpu.async_copy(ptab.at[idx_s.at[c]], pbufs[j], sems[j])
        pltpu.async_copy(qtab.at[idx_d.at[c]], qbufs[j], sems[j])

    for j in range(NSLOT):
        issue(j, j)

    def body(o, _):
        for j in range(NSLOT):
            c = o * NSLOT + j
            base = pl.multiple_of(wid * EW + c * CH, CH)
            pltpu.make_async_copy(ptab.at[idx_s.at[c]], pbufs[j], sems[j]).wait()
            pltpu.make_async_copy(qtab.at[idx_d.at[c]], qbufs[j], sems[j]).wait()
            pltpu.sync_copy(pbufs[j].at[:, pl.ds(0, OUT)], g1.at[pl.ds(base, CH)])
            pltpu.sync_copy(qbufs[j].at[:, pl.ds(0, OUT)], g2.at[pl.ds(base, CH)])
            pltpu.sync_copy(pbufs[j].at[:, pl.ds(OUT, 16)],
                            gx.at[pl.ds(base, CH), pl.ds(0, 16)])
            pltpu.sync_copy(qbufs[j].at[:, pl.ds(OUT, 16)],
                            gx.at[pl.ds(base, CH), pl.ds(16, 16)])

            @pl.when(c + NSLOT < NCHUNK)
            def _():
                issue(c + NSLOT, j)
        return _

    lax.fori_loop(0, NCHUNK // NSLOT, body, None)


def _gather(ptab, qtab, src3, dst3):
    mesh = plsc.VectorSubcoreMesh(core_axis_name="c", subcore_axis_name="s",
                                  num_cores=NC, num_subcores=NS)
    f = pl.kernel(
        _gather_body,
        out_type=[
            jax.ShapeDtypeStruct((ES, OUT), jnp.float32),
            jax.ShapeDtypeStruct((ES, OUT), jnp.float32),
            jax.ShapeDtypeStruct((ES, OUT), jnp.float32),
        ],
        mesh=mesh,
        scratch_types=(
            [pltpu.VMEM((NCHUNK, CH), jnp.int32)] * 2
            + [pltpu.VMEM((CH, TW), jnp.float32)] * (2 * NSLOT)
            + [pltpu.SemaphoreType.DMA] * NSLOT
        ),
        compiler_params=pltpu.CompilerParams(use_tc_tiling_on_sc=False),
    )
    return f(ptab, qtab, src3, dst3)


def _edge_body(invs_ref, g1_ref, g2_ref, gx_ref, eat_ref, wea_ref, wrbf_ref,
               ge1_ref, bte1_ref, we2_ref, be2_ref, wc1_ref, bc1_ref,
               wc2_ref, m_ref, m2_ref):
    pre = g1_ref[...] + g2_ref[...]
    xr = gx_ref[:, 0:3] - gx_ref[:, 16:19]
    nrm = jnp.sqrt(jnp.sum(xr * xr, axis=1, keepdims=True)) + 1.0
    xr = xr / nrm
    mag = jnp.sum(xr * xr, axis=1, keepdims=True)
    rbf = jnp.exp(-mag * invs_ref[...])
    pre = pre + lax.dot_general(eat_ref[...], wea_ref[...],
                                (((0,), (0,)), ((), ())),
                                preferred_element_type=jnp.float32)
    pre = pre + jnp.dot(rbf, wrbf_ref[...], preferred_element_type=jnp.float32)
    t = _lrelu(pre)
    t = _ln(t, ge1_ref[...], bte1_ref[...])
    m = _lrelu(jnp.dot(t, we2_ref[...], preferred_element_type=jnp.float32) + be2_ref[...])
    c1 = _lrelu(jnp.dot(m, wc1_ref[...], preferred_element_type=jnp.float32) + bc1_ref[...])
    cw = jnp.sum(c1 * wc2_ref[...], axis=1, keepdims=True)
    xupd = xr * cw
    ones = jnp.ones((TE, 1), jnp.float32)
    pad = jnp.zeros((TE, OUT - 4), jnp.float32)
    m_ref[...] = m
    m2_ref[...] = jnp.concatenate([xupd, ones, pad], axis=1)


def _edge_mlp(k, invs, g1, g2, gx, ea_t, wea, wrbf16, ge1, bte1, we2, be2,
              wc1, bc1, wc2r):
    grid = (ES // TE,)
    w0 = lambda i: (0, 0)
    blk = ES // TE
    return pl.pallas_call(
        _edge_body,
        grid=grid,
        in_specs=[
            pl.BlockSpec((1, 16), w0),
            pl.BlockSpec((TE, OUT), lambda i: (i, 0)),
            pl.BlockSpec((TE, OUT), lambda i: (i, 0)),
            pl.BlockSpec((TE, OUT), lambda i: (i, 0)),
            pl.BlockSpec((EIN, TE), lambda i, _k=k, _b=blk: (0, i + _k * _b)),
            pl.BlockSpec((EIN, OUT), w0),
            pl.BlockSpec((16, OUT), w0),
            pl.BlockSpec((1, OUT), w0),
            pl.BlockSpec((1, OUT), w0),
            pl.BlockSpec((OUT, OUT), w0),
            pl.BlockSpec((1, OUT), w0),
            pl.BlockSpec((OUT, OUT), w0),
            pl.BlockSpec((1, OUT), w0),
            pl.BlockSpec((1, OUT), w0),
        ],
        out_specs=[
            pl.BlockSpec((TE, OUT), lambda i: (i, 0)),
            pl.BlockSpec((TE, OUT), lambda i: (i, 0)),
        ],
        out_shape=[
            jax.ShapeDtypeStruct((ES, OUT), jnp.float32),
            jax.ShapeDtypeStruct((ES, OUT), jnp.float32),
        ],
    )(invs, g1, g2, gx, ea_t, wea, wrbf16, ge1, bte1, we2, be2, wc1, bc1, wc2r)


def _scatter_body(m_hbm, m2_hbm, dst3, init1, init2, s1, s2, *rest):
    idx = rest[0]
    r128 = rest[1:1 + NSLOT]
    r16 = rest[1 + NSLOT:1 + 2 * NSLOT]
    acc1, acc2 = rest[1 + 2 * NSLOT], rest[2 + 2 * NSLOT]
    sems = rest[3 + 2 * NSLOT:3 + 3 * NSLOT]
    cid = lax.axis_index("c")
    sid = lax.axis_index("s")
    wid = sid * NC + cid
    row0 = pl.multiple_of(sid * RPT, RPT)
    pltpu.sync_copy(init1.at[cid, pl.ds(row0, RPT)], acc1.at[pl.ds(row0, RPT)])
    pltpu.sync_copy(init2.at[cid, pl.ds(row0, RPT)], acc2.at[pl.ds(row0, RPT)])
    pltpu.sync_copy(dst3.at[wid], idx)
    plsc.subcore_barrier()

    def issue(c, j):
        base = pl.multiple_of(wid * EW + c * CH, CH)
        pltpu.async_copy(m_hbm.at[pl.ds(base, CH)], r128[j], sems[j])
        pltpu.async_copy(m2_hbm.at[pl.ds(base, CH), pl.ds(0, 16)], r16[j], sems[j])

    for j in range(NSLOT):
        issue(j, j)

    def body(o, _):
        for j in range(NSLOT):
            c = o * NSLOT + j
            base = pl.multiple_of(wid * EW + c * CH, CH)
            pltpu.make_async_copy(m_hbm.at[pl.ds(base, CH)], r128[j], sems[j]).wait()
            pltpu.make_async_copy(m2_hbm.at[pl.ds(base, CH), pl.ds(0, 16)],
                                  r16[j], sems[j]).wait()
            pltpu.sync_copy(r128[j], acc1.at[idx.at[c]], add=True)
            pltpu.sync_copy(r16[j], acc2.at[idx.at[c]], add=True)

            @pl.when(c + NSLOT < NCHUNK)
            def _():
                issue(c + NSLOT, j)
        return _

    lax.fori_loop(0, NCHUNK // NSLOT, body, None)
    plsc.subcore_barrier()
    pltpu.sync_copy(acc1.at[pl.ds(row0, RPT)], s1.at[cid, pl.ds(row0, RPT)])
    pltpu.sync_copy(acc2.at[pl.ds(row0, RPT)], s2.at[cid, pl.ds(row0, RPT)])


def _scatter(m, m2, dst3, init1, init2):
    mesh = plsc.VectorSubcoreMesh(core_axis_name="c", subcore_axis_name="s",
                                  num_cores=NC, num_subcores=NS)
    f = pl.kernel(
        _scatter_body,
        out_type=[
            jax.ShapeDtypeStruct((NC, NPAD, OUT), jnp.float32),
            jax.ShapeDtypeStruct((NC, NPAD, 16), jnp.float32),
        ],
        mesh=mesh,
        scratch_types=(
            [pltpu.VMEM((NCHUNK, CH), jnp.int32)]
            + [pltpu.VMEM((CH, OUT), jnp.float32)] * NSLOT
            + [pltpu.VMEM((CH, 16), jnp.float32)] * NSLOT
            + [pltpu.VMEM_SHARED((NPAD, OUT), jnp.float32),
               pltpu.VMEM_SHARED((NPAD, 16), jnp.float32)]
            + [pltpu.SemaphoreType.DMA] * NSLOT
        ),
        compiler_params=pltpu.CompilerParams(use_tc_tiling_on_sc=False),
    )
    return f(m, m2, dst3, init1, init2)


def _node_body(h_ref, orig_ref, xp_ref, s1_ref, s2_ref, wna_ref, wnb_ref,
               wnd_ref, bn1_ref, gn1_ref, btn1_ref, wn2_ref, bn2_ref,
               gnn_ref, bnn_ref, hnew_ref, xnew_ref):
    s = s1_ref[0] + s1_ref[1]
    s2 = s2_ref[0] + s2_ref[1]
    hb = h_ref[...]
    xsum = s2[:, 0:3]
    deg = jnp.maximum(s2[:, 3:4], 1.0)
    magg = s / deg
    lnh = _ln(hb, gnn_ref[...], bnn_ref[...])
    t = (jnp.dot(lnh, wna_ref[...], preferred_element_type=jnp.float32)
         + jnp.dot(magg, wnb_ref[...], preferred_element_type=jnp.float32)
         + jnp.dot(orig_ref[...], wnd_ref[...], preferred_element_type=jnp.float32)
         + bn1_ref[...])
    t = _ln(_lrelu(t), gn1_ref[...], btn1_ref[...])
    nu = jnp.dot(t, wn2_ref[...], preferred_element_type=jnp.float32) + bn2_ref[...]
    hnew_ref[...] = 0.75 * nu + 0.25 * hb
    xn = xp_ref[...][:, 0:3] + xsum / deg
    xnew_ref[...] = jnp.concatenate([xn, jnp.zeros((TN, 13), jnp.float32)], axis=1)


def _node_mlp(h, orig, xp16, s1, s2, wna, wnb, wnd, bn1, gn1, btn1, wn2, bn2,
              gnn, bnn):
    grid = (N // TN,)
    w0 = lambda i: (0, 0)
    return pl.pallas_call(
        _node_body,
        grid=grid,
        in_specs=[
            pl.BlockSpec((TN, H), lambda i: (i, 0)),
            pl.BlockSpec((TN, H), lambda i: (i, 0)),
            pl.BlockSpec((TN, 16), lambda i: (i, 0)),
            pl.BlockSpec((NC, TN, OUT), lambda i: (0, i, 0)),
            pl.BlockSpec((NC, TN, 16), lambda i: (0, i, 0)),
            pl.BlockSpec((H, H), w0),
            pl.BlockSpec((H, H), w0),
            pl.BlockSpec((H, H), w0),
            pl.BlockSpec((1, H), w0),
            pl.BlockSpec((1, H), w0),
            pl.BlockSpec((1, H), w0),
            pl.BlockSpec((H, OUT), w0),
            pl.BlockSpec((1, OUT), w0),
            pl.BlockSpec((1, H), w0),
            pl.BlockSpec((1, H), w0),
        ],
        out_specs=[
            pl.BlockSpec((TN, H), lambda i: (i, 0)),
            pl.BlockSpec((TN, 16), lambda i: (i, 0)),
        ],
        out_shape=[
            jax.ShapeDtypeStruct((N, H), jnp.float32),
            jax.ShapeDtypeStruct((N, 16), jnp.float32),
        ],
    )(h, orig, xp16, s1, s2, wna, wnb, wnd, bn1, gn1, btn1, wn2, bn2, gnn, bnn)


def kernel(h, x, orig_node_feats, edge_attr, edge_index,
           We1, be1, ge1, bte1, We2, be2,
           Wn1, bn1, gn1, btn1, Wn2, bn2,
           Wc1, bc1, Wc2, g_nn, b_nn):
    src4 = edge_index[0].astype(jnp.int32).reshape(S, NW, NCHUNK, CH)
    dst4 = edge_index[1].astype(jnp.int32).reshape(S, NW, NCHUNK, CH)
    xp16 = jnp.pad(x, ((0, 0), (0, 13)))
    ea_t = edge_attr.T

    We1a = We1[:H]
    We1b = We1[H:2 * H]
    Wea = We1[2 * H:2 * H + EIN]
    Wrbf16 = jnp.concatenate([We1[2 * H + EIN:], be1[None, :]], axis=0)
    invs = jnp.array([1.0 / s for s in SIGMAS] + [0.0], jnp.float32)[None, :]

    ptab, qtab = _prep(h, xp16, We1a, We1b)
    s1 = jnp.zeros((NC, NPAD, OUT), jnp.float32)
    s2 = jnp.zeros((NC, NPAD, 16), jnp.float32)
    for k in range(S):
        g1, g2, gx = _gather(ptab, qtab, src4[k], dst4[k])
        m, m2 = _edge_mlp(k, invs, g1, g2, gx, ea_t,
                          Wea, Wrbf16, ge1[None, :], bte1[None, :],
                          We2, be2[None, :], Wc1, bc1[None, :], Wc2.T)
        s1, s2 = _scatter(m, m2, dst4[k], s1, s2)
    h_new, xnew16 = _node_mlp(h, orig_node_feats, xp16, s1, s2,
                              Wn1[:H], Wn1[H:2 * H], Wn1[3 * H:],
                              bn1[None, :], gn1[None, :], btn1[None, :],
                              Wn2, bn2[None, :], g_nn[None, :], b_nn[None, :])
    return (h_new, xnew16[:, 0:3])


# speedup vs baseline: 1.0546x; 1.0087x over previous
import functools

import jax
import jax.numpy as jnp
from jax import lax
from jax.experimental import pallas as pl
from jax.experimental.pallas import tpu as pltpu
from jax.experimental.pallas import tpu_sc as plsc

N = 10000
E = 320000
H = 128
EIN = 16
OUT = 128
NSIG = 15
SIGMAS = [1.5 ** i for i in range(NSIG)]
TW = 144
NC, NS = 2, 16
NW = NC * NS
S = 2
ES = E // S
EW = ES // NW
CH = 40
NCHUNK = EW // CH
NSLOT = 5
NPAD = 10240
RPT = NPAD // NS
TE = 6400
TN = 2000


def _ln(v, g, b, eps=1e-5):
    mu = jnp.mean(v, axis=-1, keepdims=True)
    var = jnp.mean((v - mu) ** 2, axis=-1, keepdims=True)
    return (v - mu) * jax.lax.rsqrt(var + eps) * g + b


def _lrelu(v, s=0.01):
    return jnp.maximum(v, s * v)


def _prep_body(h_ref, xp_ref, wa_ref, wb_ref, p_ref, q_ref):
    hb = h_ref[...]
    xp = xp_ref[...]
    p_ref[...] = jnp.concatenate([jnp.dot(hb, wa_ref[...],
                                          preferred_element_type=jnp.float32), xp], axis=1)
    q_ref[...] = jnp.concatenate([jnp.dot(hb, wb_ref[...],
                                          preferred_element_type=jnp.float32), xp], axis=1)


def _prep(h, xp16, wa, wb):
    grid = (N // TN,)
    return pl.pallas_call(
        _prep_body,
        grid=grid,
        in_specs=[
            pl.BlockSpec((TN, H), lambda i: (i, 0)),
            pl.BlockSpec((TN, 16), lambda i: (i, 0)),
            pl.BlockSpec((H, OUT), lambda i: (0, 0)),
            pl.BlockSpec((H, OUT), lambda i: (0, 0)),
        ],
        out_specs=[
            pl.BlockSpec((TN, TW), lambda i: (i, 0)),
            pl.BlockSpec((TN, TW), lambda i: (i, 0)),
        ],
        out_shape=[
            jax.ShapeDtypeStruct((N, TW), jnp.float32),
            jax.ShapeDtypeStruct((N, TW), jnp.float32),
        ],
    )(h, xp16, wa, wb)


def _gather_body(k, ei5, ptab, qtab, g1, g2, gx, *rest):
    idx_s, idx_d = rest[0], rest[1]
    pbufs = rest[2:2 + NSLOT]
    qbufs = rest[2 + NSLOT:2 + 2 * NSLOT]
    sems = rest[2 + 2 * NSLOT:2 + 3 * NSLOT]
    cid = lax.axis_index("c")
    sid = lax.axis_index("s")
    wid = sid * NC + cid
    pltpu.sync_copy(ei5.at[0, k, wid], idx_s)
    pltpu.sync_copy(ei5.at[1, k, wid], idx_d)

    def issue(c, j):
        pltpu.async_copy(ptab.at[idx_s.at[c]], pbufs[j], sems[j])
        pltpu.async_copy(qtab.at[idx_d.at[c]], qbufs[j], sems[j])

    for j in range(NSLOT):
        issue(j, j)

    def body(o, _):
        for j in range(NSLOT):
            c = o * NSLOT + j
            base = pl.multiple_of(wid * EW + c * CH, CH)
            pltpu.make_async_copy(ptab.at[idx_s.at[c]], pbufs[j], sems[j]).wait()
            pltpu.make_async_copy(qtab.at[idx_d.at[c]], qbufs[j], sems[j]).wait()
            pltpu.sync_copy(pbufs[j].at[:, pl.ds(0, OUT)], g1.at[pl.ds(base, CH)])
            pltpu.sync_copy(qbufs[j].at[:, pl.ds(0, OUT)], g2.at[pl.ds(base, CH)])
            pltpu.sync_copy(pbufs[j].at[:, pl.ds(OUT, 16)],
                            gx.at[pl.ds(base, CH), pl.ds(0, 16)])
            pltpu.sync_copy(qbufs[j].at[:, pl.ds(OUT, 16)],
                            gx.at[pl.ds(base, CH), pl.ds(16, 16)])

            @pl.when(c + NSLOT < NCHUNK)
            def _():
                issue(c + NSLOT, j)
        return _

    lax.fori_loop(0, NCHUNK // NSLOT, body, None)


def _gather(k, ei5, ptab, qtab):
    mesh = plsc.VectorSubcoreMesh(core_axis_name="c", subcore_axis_name="s",
                                  num_cores=NC, num_subcores=NS)
    f = pl.kernel(
        functools.partial(_gather_body, k),
        out_type=[
            jax.ShapeDtypeStruct((ES, OUT), jnp.float32),
            jax.ShapeDtypeStruct((ES, OUT), jnp.float32),
            jax.ShapeDtypeStruct((ES, OUT), jnp.float32),
        ],
        mesh=mesh,
        scratch_types=(
            [pltpu.VMEM((NCHUNK, CH), jnp.int32)] * 2
            + [pltpu.VMEM((CH, TW), jnp.float32)] * (2 * NSLOT)
            + [pltpu.SemaphoreType.DMA] * NSLOT
        ),
        compiler_params=pltpu.CompilerParams(use_tc_tiling_on_sc=False),
    )
    return f(ei5, ptab, qtab)


def _edge_body(invs_ref, g1_ref, g2_ref, gx_ref, eat_ref, wea_ref, wrbf_ref,
               ge1_ref, bte1_ref, we2_ref, be2_ref, wc1_ref, bc1_ref,
               wc2_ref, m_ref, m2_ref):
    pre = g1_ref[...] + g2_ref[...]
    xr = gx_ref[:, 0:3] - gx_ref[:, 16:19]
    nrm = jnp.sqrt(jnp.sum(xr * xr, axis=1, keepdims=True)) + 1.0
    xr = xr / nrm
    mag = jnp.sum(xr * xr, axis=1, keepdims=True)
    rbf = jnp.exp(-mag * invs_ref[...])
    pre = pre + lax.dot_general(eat_ref[...], wea_ref[...],
                                (((0,), (0,)), ((), ())),
                                preferred_element_type=jnp.float32)
    pre = pre + jnp.dot(rbf, wrbf_ref[...], preferred_element_type=jnp.float32)
    t = _lrelu(pre)
    t = _ln(t, ge1_ref[...], bte1_ref[...])
    m = _lrelu(jnp.dot(t, we2_ref[...], preferred_element_type=jnp.float32) + be2_ref[...])
    c1 = _lrelu(jnp.dot(m, wc1_ref[...], preferred_element_type=jnp.float32) + bc1_ref[...])
    cw = jnp.sum(c1 * wc2_ref[...], axis=1, keepdims=True)
    xupd = xr * cw
    ones = jnp.ones((TE, 1), jnp.float32)
    pad = jnp.zeros((TE, OUT - 4), jnp.float32)
    m_ref[...] = m
    m2_ref[...] = jnp.concatenate([xupd, ones, pad], axis=1)


def _edge_mlp(k, invs, g1, g2, gx, ea_t, wea, wrbf16, ge1, bte1, we2, be2,
              wc1, bc1, wc2r):
    grid = (ES // TE,)
    w0 = lambda i: (0, 0)
    blk = ES // TE
    return pl.pallas_call(
        _edge_body,
        grid=grid,
        in_specs=[
            pl.BlockSpec((1, 16), w0),
            pl.BlockSpec((TE, OUT), lambda i: (i, 0)),
            pl.BlockSpec((TE, OUT), lambda i: (i, 0)),
            pl.BlockSpec((TE, OUT), lambda i: (i, 0)),
            pl.BlockSpec((EIN, TE), lambda i, _k=k, _b=blk: (0, i + _k * _b)),
            pl.BlockSpec((EIN, OUT), w0),
            pl.BlockSpec((16, OUT), w0),
            pl.BlockSpec((1, OUT), w0),
            pl.BlockSpec((1, OUT), w0),
            pl.BlockSpec((OUT, OUT), w0),
            pl.BlockSpec((1, OUT), w0),
            pl.BlockSpec((OUT, OUT), w0),
            pl.BlockSpec((1, OUT), w0),
            pl.BlockSpec((1, OUT), w0),
        ],
        out_specs=[
            pl.BlockSpec((TE, OUT), lambda i: (i, 0)),
            pl.BlockSpec((TE, OUT), lambda i: (i, 0)),
        ],
        out_shape=[
            jax.ShapeDtypeStruct((ES, OUT), jnp.float32),
            jax.ShapeDtypeStruct((ES, OUT), jnp.float32),
        ],
    )(invs, g1, g2, gx, ea_t, wea, wrbf16, ge1, bte1, we2, be2, wc1, bc1, wc2r)


def _scatter_body(k, m_hbm, m2_hbm, ei5, init1, init2, s1, s2, *rest):
    idx = rest[0]
    r128 = rest[1:1 + NSLOT]
    r16 = rest[1 + NSLOT:1 + 2 * NSLOT]
    acc1, acc2 = rest[1 + 2 * NSLOT], rest[2 + 2 * NSLOT]
    sems = rest[3 + 2 * NSLOT:3 + 3 * NSLOT]
    cid = lax.axis_index("c")
    sid = lax.axis_index("s")
    wid = sid * NC + cid
    row0 = pl.multiple_of(sid * RPT, RPT)
    pltpu.sync_copy(init1.at[cid, pl.ds(row0, RPT)], acc1.at[pl.ds(row0, RPT)])
    pltpu.sync_copy(init2.at[cid, pl.ds(row0, RPT)], acc2.at[pl.ds(row0, RPT)])
    pltpu.sync_copy(ei5.at[1, k, wid], idx)
    plsc.subcore_barrier()

    def issue(c, j):
        base = pl.multiple_of(wid * EW + c * CH, CH)
        pltpu.async_copy(m_hbm.at[pl.ds(base, CH)], r128[j], sems[j])
        pltpu.async_copy(m2_hbm.at[pl.ds(base, CH), pl.ds(0, 16)], r16[j], sems[j])

    for j in range(NSLOT):
        issue(j, j)

    def body(o, _):
        for j in range(NSLOT):
            c = o * NSLOT + j
            base = pl.multiple_of(wid * EW + c * CH, CH)
            pltpu.make_async_copy(m_hbm.at[pl.ds(base, CH)], r128[j], sems[j]).wait()
            pltpu.make_async_copy(m2_hbm.at[pl.ds(base, CH), pl.ds(0, 16)],
                                  r16[j], sems[j]).wait()
            pltpu.sync_copy(r128[j], acc1.at[idx.at[c]], add=True)
            pltpu.sync_copy(r16[j], acc2.at[idx.at[c]], add=True)

            @pl.when(c + NSLOT < NCHUNK)
            def _():
                issue(c + NSLOT, j)
        return _

    lax.fori_loop(0, NCHUNK // NSLOT, body, None)
    plsc.subcore_barrier()
    pltpu.sync_copy(acc1.at[pl.ds(row0, RPT)], s1.at[cid, pl.ds(row0, RPT)])
    pltpu.sync_copy(acc2.at[pl.ds(row0, RPT)], s2.at[cid, pl.ds(row0, RPT)])


def _scatter(k, m, m2, ei5, init1, init2):
    mesh = plsc.VectorSubcoreMesh(core_axis_name="c", subcore_axis_name="s",
                                  num_cores=NC, num_subcores=NS)
    f = pl.kernel(
        functools.partial(_scatter_body, k),
        out_type=[
            jax.ShapeDtypeStruct((NC, NPAD, OUT), jnp.float32),
            jax.ShapeDtypeStruct((NC, NPAD, 16), jnp.float32),
        ],
        mesh=mesh,
        scratch_types=(
            [pltpu.VMEM((NCHUNK, CH), jnp.int32)]
            + [pltpu.VMEM((CH, OUT), jnp.float32)] * NSLOT
            + [pltpu.VMEM((CH, 16), jnp.float32)] * NSLOT
            + [pltpu.VMEM_SHARED((NPAD, OUT), jnp.float32),
               pltpu.VMEM_SHARED((NPAD, 16), jnp.float32)]
            + [pltpu.SemaphoreType.DMA] * NSLOT
        ),
        compiler_params=pltpu.CompilerParams(use_tc_tiling_on_sc=False),
    )
    return f(m, m2, ei5, init1, init2)


def _node_body(h_ref, orig_ref, xp_ref, s1_ref, s2_ref, wna_ref, wnb_ref,
               wnd_ref, bn1_ref, gn1_ref, btn1_ref, wn2_ref, bn2_ref,
               gnn_ref, bnn_ref, hnew_ref, xnew_ref):
    s = s1_ref[0] + s1_ref[1]
    s2 = s2_ref[0] + s2_ref[1]
    hb = h_ref[...]
    xsum = s2[:, 0:3]
    deg = jnp.maximum(s2[:, 3:4], 1.0)
    magg = s / deg
    lnh = _ln(hb, gnn_ref[...], bnn_ref[...])
    t = (jnp.dot(lnh, wna_ref[...], preferred_element_type=jnp.float32)
         + jnp.dot(magg, wnb_ref[...], preferred_element_type=jnp.float32)
         + jnp.dot(orig_ref[...], wnd_ref[...], preferred_element_type=jnp.float32)
         + bn1_ref[...])
    t = _ln(_lrelu(t), gn1_ref[...], btn1_ref[...])
    nu = jnp.dot(t, wn2_ref[...], preferred_element_type=jnp.float32) + bn2_ref[...]
    hnew_ref[...] = 0.75 * nu + 0.25 * hb
    xn = xp_ref[...][:, 0:3] + xsum / deg
    xnew_ref[...] = jnp.concatenate([xn, jnp.zeros((TN, 13), jnp.float32)], axis=1)


def _node_mlp(h, orig, xp16, s1, s2, wna, wnb, wnd, bn1, gn1, btn1, wn2, bn2,
              gnn, bnn):
    grid = (N // TN,)
    w0 = lambda i: (0, 0)
    return pl.pallas_call(
        _node_body,
        grid=grid,
        in_specs=[
            pl.BlockSpec((TN, H), lambda i: (i, 0)),
            pl.BlockSpec((TN, H), lambda i: (i, 0)),
            pl.BlockSpec((TN, 16), lambda i: (i, 0)),
            pl.BlockSpec((NC, TN, OUT), lambda i: (0, i, 0)),
            pl.BlockSpec((NC, TN, 16), lambda i: (0, i, 0)),
            pl.BlockSpec((H, H), w0),
            pl.BlockSpec((H, H), w0),
            pl.BlockSpec((H, H), w0),
            pl.BlockSpec((1, H), w0),
            pl.BlockSpec((1, H), w0),
            pl.BlockSpec((1, H), w0),
            pl.BlockSpec((H, OUT), w0),
            pl.BlockSpec((1, OUT), w0),
            pl.BlockSpec((1, H), w0),
            pl.BlockSpec((1, H), w0),
        ],
        out_specs=[
            pl.BlockSpec((TN, H), lambda i: (i, 0)),
            pl.BlockSpec((TN, 16), lambda i: (i, 0)),
        ],
        out_shape=[
            jax.ShapeDtypeStruct((N, H), jnp.float32),
            jax.ShapeDtypeStruct((N, 16), jnp.float32),
        ],
    )(h, orig, xp16, s1, s2, wna, wnb, wnd, bn1, gn1, btn1, wn2, bn2, gnn, bnn)


def kernel(h, x, orig_node_feats, edge_attr, edge_index,
           We1, be1, ge1, bte1, We2, be2,
           Wn1, bn1, gn1, btn1, Wn2, bn2,
           Wc1, bc1, Wc2, g_nn, b_nn):
    ei5 = edge_index.astype(jnp.int32).reshape(2, S, NW, NCHUNK, CH)
    xp16 = jnp.pad(x, ((0, 0), (0, 13)))
    ea_t = edge_attr.T

    We1a = We1[:H]
    We1b = We1[H:2 * H]
    Wea = We1[2 * H:2 * H + EIN]
    Wrbf16 = jnp.concatenate([We1[2 * H + EIN:], be1[None, :]], axis=0)
    invs = jnp.array([1.0 / s for s in SIGMAS] + [0.0], jnp.float32)[None, :]

    ptab, qtab = _prep(h, xp16, We1a, We1b)
    s1 = jnp.zeros((NC, NPAD, OUT), jnp.float32)
    s2 = jnp.zeros((NC, NPAD, 16), jnp.float32)
    for k in range(S):
        g1, g2, gx = _gather(k, ei5, ptab, qtab)
        m, m2 = _edge_mlp(k, invs, g1, g2, gx, ea_t,
                          Wea, Wrbf16, ge1[None, :], bte1[None, :],
                          We2, be2[None, :], Wc1, bc1[None, :], Wc2.T)
        s1, s2 = _scatter(k, m, m2, ei5, s1, s2)
    h_new, xnew16 = _node_mlp(h, orig_node_feats, xp16, s1, s2,
                              Wn1[:H], Wn1[H:2 * H], Wn1[3 * H:],
                              bn1[None, :], gn1[None, :], btn1[None, :],
                              Wn2, bn2[None, :], g_nn[None, :], b_nn[None, :])
    return (h_new, xnew16[:, 0:3])
